# baseline (speedup 1.0000x reference)
import jax
import jax.numpy as jnp
from jax.experimental import pallas as pl

R_BLK = 256


def _add_pe_kernel(x_ref, pe_ref, o_ref):
    blk, B = x_ref.shape
    x3 = x_ref[...].reshape(blk // 128, 128, B)
    o_ref[...] = (x3 + pe_ref[0][:, :, None]).reshape(blk, B)


def kernel(x, pos_emb):
    B, S, D = x.shape
    xt = x.transpose(1, 2, 0).reshape(S * D, B)
    pe = pos_emb[:S].reshape(S * D // R_BLK, R_BLK // 128, 128)
    out = pl.pallas_call(
        _add_pe_kernel,
        grid=(S * D // R_BLK,),
        in_specs=[
            pl.BlockSpec((R_BLK, B), lambda i: (i, 0)),
            pl.BlockSpec((1, R_BLK // 128, 128), lambda i: (i, 0, 0)),
        ],
        out_specs=pl.BlockSpec((R_BLK, B), lambda i: (i, 0)),
        out_shape=jax.ShapeDtypeStruct((S * D, B), x.dtype),
    )(xt, pe)
    return out.reshape(S, D, B).transpose(2, 0, 1)


# speedup vs baseline: 1.0186x; 1.0186x over previous
import jax
import jax.numpy as jnp
from jax.experimental import pallas as pl

R_BLK = 640


def _add_pe_kernel(x_ref, pe_ref, o_ref):
    blk, B = x_ref.shape
    x3 = x_ref[...].reshape(blk // 128, 128, B)
    o_ref[...] = (x3 + pe_ref[0][:, :, None]).reshape(blk, B)


def kernel(x, pos_emb):
    B, S, D = x.shape
    xt = x.transpose(1, 2, 0).reshape(S * D, B)
    pe = pos_emb[:S].reshape(S * D // R_BLK, R_BLK // 128, 128)
    out = pl.pallas_call(
        _add_pe_kernel,
        grid=(S * D // R_BLK,),
        in_specs=[
            pl.BlockSpec((R_BLK, B), lambda i: (i, 0)),
            pl.BlockSpec((1, R_BLK // 128, 128), lambda i: (i, 0, 0)),
        ],
        out_specs=pl.BlockSpec((R_BLK, B), lambda i: (i, 0)),
        out_shape=jax.ShapeDtypeStruct((S * D, B), x.dtype),
    )(xt, pe)
    return out.reshape(S, D, B).transpose(2, 0, 1)
